# baseline (speedup 1.0000x reference)
import functools

import jax
import jax.numpy as jnp
import numpy as np
from jax import lax
from jax.experimental import pallas as pl
from jax.experimental.pallas import tpu as pltpu
from jax.experimental.pallas import tpu_sc as plsc

_VOCAB = 100000
_D = 1024
_B = 4
_S = 4096
_SCALE = np.float32(1.0 / np.sqrt(_D))

_NC = 2
_NS = 16
_NW = _NC * _NS
_L = 16

_N = _B * _S
_PER_W = _N // _NW
_CH = 32
_NCH = _PER_W // _CH


def _pos_encoding() -> np.ndarray:
    pos = np.arange(_S, dtype=np.float32)[:, None]
    div = np.exp(
        np.arange(0, _D, 2, dtype=np.float32) * (-np.log(10000.0) / _D)
    )
    pe = np.zeros((_S, _D), dtype=np.float32)
    pe[:, 0::2] = np.sin(pos * div)
    pe[:, 1::2] = np.cos(pos * div)
    return pe


_PE = _pos_encoding()


def _sc_body(idx_hbm, pe_hbm, table_hbm, out_hbm, idx_v, pe_v, x_v, sem):
    cid = lax.axis_index("c")
    sid = lax.axis_index("s")
    wid = sid * _NC + cid
    base = wid * _PER_W
    s_base = lax.rem(base, _S)

    pltpu.sync_copy(idx_hbm.at[pl.ds(base, _PER_W)], idx_v)

    def chunk_body(ch, carry):
        off = ch * _CH
        pltpu.sync_copy(pe_hbm.at[pl.ds(s_base + off, _CH)], pe_v)
        pltpu.async_copy(
            table_hbm.at[idx_v.at[pl.ds(off, _CH)]], x_v, sem
        ).wait()

        def row_body(r, rcarry):
            for c in range(_D // _L):
                sl = pl.ds(c * _L, _L)
                x_v[r, sl] = x_v[r, sl] * _SCALE + pe_v[r, sl]
            return rcarry

        lax.fori_loop(0, _CH, row_body, 0)
        pltpu.sync_copy(x_v, out_hbm.at[pl.ds(base + off, _CH)])
        return carry

    lax.fori_loop(0, _NCH, chunk_body, 0)


@jax.jit
def _embed(idx_flat, table, pe):
    fn = functools.partial(
        pl.kernel,
        mesh=plsc.VectorSubcoreMesh(core_axis_name="c", subcore_axis_name="s"),
        out_type=jax.ShapeDtypeStruct((_N, _D), jnp.float32),
        scratch_types=[
            pltpu.VMEM((_PER_W,), jnp.int32),
            pltpu.VMEM((_CH, _D), jnp.float32),
            pltpu.VMEM((_CH, _D), jnp.float32),
            pltpu.SemaphoreType.DMA,
        ],
    )(_sc_body)
    return fn(idx_flat, pe, table)


def kernel(inputs, table):
    idx_flat = inputs.reshape(_N)
    pe = jnp.asarray(_PE)
    out = _embed(idx_flat, table, pe)
    return out.reshape(_B, _S, _D)


# speedup vs baseline: 1.9171x; 1.9171x over previous
import functools

import jax
import jax.numpy as jnp
import numpy as np
from jax import lax
from jax.experimental import pallas as pl
from jax.experimental.pallas import tpu as pltpu
from jax.experimental.pallas import tpu_sc as plsc

_VOCAB = 100000
_D = 1024
_B = 4
_S = 4096
_SCALE = np.float32(1.0 / np.sqrt(_D))

_NC = 2
_NS = 16
_NW = _NC * _NS
_L = 16

_N = _B * _S
_SPW = _S // _NW
_R = 8
_NCH = _SPW // _R
_NT = _NCH // 2


def _pos_encoding() -> np.ndarray:
    pos = np.arange(_S, dtype=np.float32)[:, None]
    div = np.exp(
        np.arange(0, _D, 2, dtype=np.float32) * (-np.log(10000.0) / _D)
    )
    pe = np.zeros((_S, _D), dtype=np.float32)
    pe[:, 0::2] = np.sin(pos * div)
    pe[:, 1::2] = np.cos(pos * div)
    return pe


_PE = _pos_encoding()


def _sc_body(idx_hbm, pe_hbm, table_hbm, out_hbm,
             idx_v, x0, x1, pe0, pe1,
             g0, g1, sc0, sc1, ps0, ps1):
    cid = lax.axis_index("c")
    sid = lax.axis_index("s")
    wid = sid * _NC + cid
    s0 = wid * _SPW

    for b in range(_B):
        pltpu.sync_copy(idx_hbm.at[pl.ds(b * _S + s0, _SPW)],
                        idx_v.at[pl.ds(b * _SPW, _SPW)])

    def fire(c, xb, peb, gsem, psem):
        pltpu.async_copy(pe_hbm.at[pl.ds(s0 + c * _R, _R)], peb, psem)
        for b in range(_B):
            pltpu.async_copy(
                table_hbm.at[idx_v.at[pl.ds(b * _SPW + c * _R, _R)]],
                xb.at[b], gsem)

    def process(c, xb, peb, gsem, psem, ssem):
        pltpu.make_async_copy(pe_hbm.at[pl.ds(s0 + c * _R, _R)],
                              peb, psem).wait()
        for b in range(_B):
            pltpu.make_async_copy(
                table_hbm.at[idx_v.at[pl.ds(b * _SPW + c * _R, _R)]],
                xb.at[b], gsem).wait()
        for b in range(_B):
            def row_body(r, carry):
                for col in range(_D // _L):
                    sl = pl.ds(col * _L, _L)
                    xb[b, r, sl] = xb[b, r, sl] * _SCALE + peb[r, sl]
                return carry
            lax.fori_loop(0, _R, row_body, 0)
            pltpu.async_copy(
                xb.at[b], out_hbm.at[pl.ds(b * _S + s0 + c * _R, _R)], ssem)

    def drain_scatter(c, xb, ssem):
        for b in range(_B):
            pltpu.make_async_copy(
                xb.at[b], out_hbm.at[pl.ds(b * _S + s0 + c * _R, _R)],
                ssem).wait()

    fire(0, x0, pe0, g0, ps0)
    fire(1, x1, pe1, g1, ps1)

    def iter_body(t, carry):
        ca = 2 * t
        cb = 2 * t + 1

        process(ca, x0, pe0, g0, ps0, sc0)

        @pl.when(t < _NT - 1)
        def _():
            drain_scatter(ca, x0, sc0)
            fire(ca + 2, x0, pe0, g0, ps0)

        process(cb, x1, pe1, g1, ps1, sc1)

        @pl.when(t < _NT - 1)
        def _():
            drain_scatter(cb, x1, sc1)
            fire(cb + 2, x1, pe1, g1, ps1)

        return carry

    lax.fori_loop(0, _NT, iter_body, 0)

    drain_scatter(2 * _NT - 2, x0, sc0)
    drain_scatter(2 * _NT - 1, x1, sc1)


@jax.jit
def _embed(idx_flat, table, pe):
    fn = functools.partial(
        pl.kernel,
        mesh=plsc.VectorSubcoreMesh(core_axis_name="c", subcore_axis_name="s"),
        out_type=jax.ShapeDtypeStruct((_N, _D), jnp.float32),
        scratch_types=[
            pltpu.VMEM((_B * _SPW,), jnp.int32),
            pltpu.VMEM((_B, _R, _D), jnp.float32),
            pltpu.VMEM((_B, _R, _D), jnp.float32),
            pltpu.VMEM((_R, _D), jnp.float32),
            pltpu.VMEM((_R, _D), jnp.float32),
            pltpu.SemaphoreType.DMA,
            pltpu.SemaphoreType.DMA,
            pltpu.SemaphoreType.DMA,
            pltpu.SemaphoreType.DMA,
            pltpu.SemaphoreType.DMA,
            pltpu.SemaphoreType.DMA,
        ],
    )(_sc_body)
    return fn(idx_flat, pe, table)


def kernel(inputs, table):
    idx_flat = inputs.reshape(_N)
    pe = jnp.asarray(_PE)
    out = _embed(idx_flat, table, pe)
    return out.reshape(_B, _S, _D)
